# Initial kernel scaffold; baseline (speedup 1.0000x reference)
import jax
import jax.numpy as jnp
from jax.experimental import pallas as pl


def kernel(x, edge_index, edge_attr, batch, emb, Wf1, bf1, Ws1, bs1, Wf2, bf2, Ws2, bs2, Wf3, bf3, Ws3, bs3, Wl, bl):
    raise NotImplementedError("write your pallas kernel here")


# speedup vs baseline: 2.5146x; 2.5146x over previous
import functools
import jax
import jax.numpy as jnp
from jax import lax
from jax.experimental import pallas as pl
from jax.experimental.pallas import tpu as pltpu
from jax.experimental.pallas import tpu_sc as plsc

N = 10000
E = 320000
H = 128
R = 32
G = 64
NVOC = 100

NC = 2
NS = 16
NW = NC * NS
EPW = E // NW
CH = 80
NCHUNK = EPW // CH
NPAD = 10240
RPT = NPAD // NS

_mesh = plsc.VectorSubcoreMesh(core_axis_name="c", subcore_axis_name="s")


@functools.partial(
    pl.kernel,
    mesh=_mesh,
    out_type=[
        jax.ShapeDtypeStruct((E, H), jnp.float32),
        jax.ShapeDtypeStruct((E, H), jnp.float32),
    ],
    scratch_types=[
        pltpu.VMEM((CH,), jnp.int32),
        pltpu.VMEM((CH,), jnp.int32),
        pltpu.VMEM((CH, H), jnp.float32),
        pltpu.VMEM((CH, H), jnp.float32),
        pltpu.SemaphoreType.DMA,
        pltpu.SemaphoreType.DMA,
    ],
)
def _sc_gather(h_hbm, src_hbm, dst_hbm, zs_hbm, zd_hbm, si, di, sr, dr, sem_s, sem_d):
    wid = lax.axis_index("s") * NC + lax.axis_index("c")
    base = wid * EPW

    def body(i, carry):
        off = base + i * CH
        pltpu.sync_copy(src_hbm.at[pl.ds(off, CH)], si)
        pltpu.sync_copy(dst_hbm.at[pl.ds(off, CH)], di)
        cs = pltpu.async_copy(h_hbm.at[si], sr, sem_s)
        cd = pltpu.async_copy(h_hbm.at[di], dr, sem_d)
        cs.wait()
        cd.wait()
        pltpu.sync_copy(sr, zs_hbm.at[pl.ds(off, CH)])
        pltpu.sync_copy(dr, zd_hbm.at[pl.ds(off, CH)])
        return carry

    lax.fori_loop(0, NCHUNK, body, 0)


@functools.partial(
    pl.kernel,
    mesh=_mesh,
    out_type=jax.ShapeDtypeStruct((NC, NPAD, H), jnp.float32),
    scratch_types=[
        pltpu.VMEM((CH,), jnp.int32),
        pltpu.VMEM((CH, H), jnp.float32),
        pltpu.VMEM_SHARED((NPAD, H), jnp.float32),
    ],
)
def _sc_scatter_add(m_hbm, dst_hbm, zero_hbm, agg_hbm, di, mr, acc):
    c = lax.axis_index("c")
    s = lax.axis_index("s")
    wid = c * NS + s
    base = wid * EPW
    r0 = s * RPT

    pltpu.sync_copy(zero_hbm.at[pl.ds(r0, RPT)], acc.at[pl.ds(r0, RPT)])
    plsc.subcore_barrier()

    def body(i, carry):
        off = base + i * CH
        pltpu.sync_copy(dst_hbm.at[pl.ds(off, CH)], di)
        pltpu.sync_copy(m_hbm.at[pl.ds(off, CH)], mr)
        pltpu.sync_copy(mr, acc.at[di], add=True)
        return carry

    lax.fori_loop(0, NCHUNK, body, 0)
    plsc.subcore_barrier()
    pltpu.sync_copy(acc.at[pl.ds(r0, RPT)], agg_hbm.at[c].at[pl.ds(r0, RPT)])


BN = 2000
GN = N // BN
BE = 2000
GE = E // BE


def _emb_body(x_ref, emb_ref, h_ref):
    xb = x_ref[0, 0, :]
    onehot = (xb[:, None] == lax.broadcasted_iota(jnp.int32, (BN, NVOC), 1))
    h_ref[...] = jnp.dot(onehot.astype(jnp.float32), emb_ref[...],
                         preferred_element_type=jnp.float32)


_emb_call = pl.pallas_call(
    _emb_body,
    grid=(GN,),
    in_specs=[
        pl.BlockSpec((1, 1, BN), lambda i: (i, 0, 0)),
        pl.BlockSpec((NVOC, H), lambda i: (0, 0)),
    ],
    out_specs=pl.BlockSpec((BN, H), lambda i: (i, 0)),
    out_shape=jax.ShapeDtypeStruct((N, H), jnp.float32),
)


def _edge_body(zd_ref, zs_ref, ea_ref, wf_ref, bf_ref, ws_ref, bs_ref, m_ref):
    zd = zd_ref[...]
    zs = zs_ref[...]
    ea = ea_ref[...]
    f = (jnp.dot(zd, wf_ref[0:H, :], preferred_element_type=jnp.float32)
         + jnp.dot(zs, wf_ref[H:2 * H, :], preferred_element_type=jnp.float32)
         + jnp.dot(ea, wf_ref[2 * H:, :], preferred_element_type=jnp.float32)
         + bf_ref[...])
    s = (jnp.dot(zd, ws_ref[0:H, :], preferred_element_type=jnp.float32)
         + jnp.dot(zs, ws_ref[H:2 * H, :], preferred_element_type=jnp.float32)
         + jnp.dot(ea, ws_ref[2 * H:, :], preferred_element_type=jnp.float32)
         + bs_ref[...])
    sig = 1.0 / (1.0 + jnp.exp(-f))
    sp = jnp.maximum(s, 0.0) + jnp.log(1.0 + jnp.exp(-jnp.abs(s)))
    m_ref[...] = sig * sp


_edge_call = pl.pallas_call(
    _edge_body,
    grid=(GE,),
    in_specs=[
        pl.BlockSpec((BE, H), lambda i: (i, 0)),
        pl.BlockSpec((BE, H), lambda i: (i, 0)),
        pl.BlockSpec((BE, R), lambda i: (i, 0)),
        pl.BlockSpec((2 * H + R, H), lambda i: (0, 0)),
        pl.BlockSpec((H,), lambda i: (0,)),
        pl.BlockSpec((2 * H + R, H), lambda i: (0, 0)),
        pl.BlockSpec((H,), lambda i: (0,)),
    ],
    out_specs=pl.BlockSpec((BE, H), lambda i: (i, 0)),
    out_shape=jax.ShapeDtypeStruct((E, H), jnp.float32),
)


def _combine_body(h_ref, a0_ref, a1_ref, o_ref):
    o_ref[...] = jnp.maximum(h_ref[...] + a0_ref[...] + a1_ref[...], 0.0)


_combine_call = pl.pallas_call(
    _combine_body,
    grid=(GN,),
    in_specs=[
        pl.BlockSpec((BN, H), lambda i: (i, 0)),
        pl.BlockSpec((BN, H), lambda i: (i, 0)),
        pl.BlockSpec((BN, H), lambda i: (i, 0)),
    ],
    out_specs=pl.BlockSpec((BN, H), lambda i: (i, 0)),
    out_shape=jax.ShapeDtypeStruct((N, H), jnp.float32),
)


def _pool_body(h_ref, a0_ref, a1_ref, b_ref, wl_ref, bl_ref, o_ref, sums, cnts):
    i = pl.program_id(0)

    @pl.when(i == 0)
    def _():
        sums[...] = jnp.zeros_like(sums)
        cnts[...] = jnp.zeros_like(cnts)

    h3 = jnp.maximum(h_ref[...] + a0_ref[...] + a1_ref[...], 0.0)
    bb = b_ref[0, 0, :]
    onehot = (bb[:, None] == lax.broadcasted_iota(jnp.int32, (BN, G), 1)).astype(jnp.float32)
    sums[...] += lax.dot_general(onehot, h3, (((0,), (0,)), ((), ())),
                                 preferred_element_type=jnp.float32)
    cnts[...] += jnp.broadcast_to(jnp.sum(onehot, axis=0)[:, None], (G, H))

    @pl.when(i == GN - 1)
    def _():
        pooled = sums[...] / jnp.maximum(cnts[...], 1.0)
        o_ref[...] = jnp.dot(pooled, wl_ref[...],
                             preferred_element_type=jnp.float32) + bl_ref[...]


_pool_call = pl.pallas_call(
    _pool_body,
    grid=(GN,),
    in_specs=[
        pl.BlockSpec((BN, H), lambda i: (i, 0)),
        pl.BlockSpec((BN, H), lambda i: (i, 0)),
        pl.BlockSpec((BN, H), lambda i: (i, 0)),
        pl.BlockSpec((1, 1, BN), lambda i: (i, 0, 0)),
        pl.BlockSpec((H, H), lambda i: (0, 0)),
        pl.BlockSpec((H,), lambda i: (0,)),
    ],
    out_specs=pl.BlockSpec((G, H), lambda i: (0, 0)),
    out_shape=jax.ShapeDtypeStruct((G, H), jnp.float32),
    scratch_shapes=[
        pltpu.VMEM((G, H), jnp.float32),
        pltpu.VMEM((G, H), jnp.float32),
    ],
)


def kernel(x, edge_index, edge_attr, batch, emb,
           Wf1, bf1, Ws1, bs1, Wf2, bf2, Ws2, bs2, Wf3, bf3, Ws3, bs3, Wl, bl):
    src = edge_index[0].astype(jnp.int32)
    dst = edge_index[1].astype(jnp.int32)
    x3 = x.reshape(GN, 1, BN).astype(jnp.int32)
    b3 = batch.reshape(GN, 1, BN).astype(jnp.int32)
    zero = jnp.zeros((NPAD, H), jnp.float32)

    h = _emb_call(x3, emb)
    layers = [(Wf1, bf1, Ws1, bs1), (Wf2, bf2, Ws2, bs2), (Wf3, bf3, Ws3, bs3)]
    agg = None
    for li, (Wf, bf, Ws, bs) in enumerate(layers):
        if li > 0:
            h = _combine_call(h, agg[0, :N], agg[1, :N])
        zs, zd = _sc_gather(h, src, dst)
        m = _edge_call(zd, zs, edge_attr, Wf, bf, Ws, bs)
        agg = _sc_scatter_add(m, dst, zero)

    return _pool_call(h, agg[0, :N], agg[1, :N], b3, Wl, bl)


# speedup vs baseline: 2.8126x; 1.1185x over previous
import functools
import jax
import jax.numpy as jnp
from jax import lax
from jax.experimental import pallas as pl
from jax.experimental.pallas import tpu as pltpu
from jax.experimental.pallas import tpu_sc as plsc

N = 10000
E = 320000
H = 128
R = 32
G = 64
NVOC = 100

NC = 2
NS = 16
NW = NC * NS
EPW = E // NW
CH = 80
NCHUNK = EPW // CH
NPAD = 10240
RPT = NPAD // NS

_mesh = plsc.VectorSubcoreMesh(core_axis_name="c", subcore_axis_name="s")


SPW = E // NS
SUP = 400
NSUB = SUP // CH
NSUP = SPW // SUP


@functools.partial(
    pl.kernel,
    mesh=_mesh,
    out_type=jax.ShapeDtypeStruct((2, E, H), jnp.float32),
    scratch_types=[
        pltpu.VMEM((SPW,), jnp.int32),
        pltpu.VMEM((SUP, H), jnp.float32),
        pltpu.VMEM((SUP, H), jnp.float32),
        pltpu.SemaphoreType.DMA,
        pltpu.SemaphoreType.DMA,
        pltpu.SemaphoreType.DMA,
        pltpu.SemaphoreType.DMA,
    ],
)
def _sc_gather(h_hbm, src_hbm, dst_hbm, z_hbm, idx, bufa, bufb, gsa, gsb, wsa, wsb):
    c = lax.axis_index("c")
    s = lax.axis_index("s")
    base = s * SPW

    @pl.when(c == 0)
    def _():
        pltpu.sync_copy(src_hbm.at[pl.ds(base, SPW)], idx)

    @pl.when(c == 1)
    def _():
        pltpu.sync_copy(dst_hbm.at[pl.ds(base, SPW)], idx)

    def issue_gathers(t, buf, gsem):
        for k in range(NSUB):
            pltpu.async_copy(
                h_hbm.at[idx.at[pl.ds(t * SUP + k * CH, CH)]],
                buf.at[pl.ds(k * CH, CH)], gsem)

    def wait_gathers(buf, gsem):
        pltpu.make_async_copy(h_hbm.at[pl.ds(0, SUP)], buf, gsem).wait()

    def issue_write(t, buf, wsem):
        pltpu.async_copy(buf, z_hbm.at[c].at[pl.ds(base + t * SUP, SUP)], wsem)

    def wait_write(buf, wsem):
        pltpu.make_async_copy(buf, z_hbm.at[c].at[pl.ds(base, SUP)], wsem).wait()

    issue_gathers(0, bufa, gsa)

    def phase(t, buf, gsem, wsem, obuf, ogsem, owsem):
        @pl.when(t >= 1)
        def _():
            wait_write(obuf, owsem)

        @pl.when(t + 1 < NSUP)
        def _():
            issue_gathers(t + 1, obuf, ogsem)

        wait_gathers(buf, gsem)
        issue_write(t, buf, wsem)

    def body(o, carry):
        phase(2 * o, bufa, gsa, wsa, bufb, gsb, wsb)
        phase(2 * o + 1, bufb, gsb, wsb, bufa, gsa, wsa)
        return carry

    lax.fori_loop(0, NSUP // 2, body, 0)
    wait_write(bufb, wsb)


MSUP = 80
MSUB = MSUP // CH
NMSUP = EPW // MSUP


@functools.partial(
    pl.kernel,
    mesh=_mesh,
    out_type=jax.ShapeDtypeStruct((NC, NPAD, H), jnp.float32),
    scratch_types=[
        pltpu.VMEM((NCHUNK, CH), jnp.int32),
        pltpu.VMEM((MSUP, H), jnp.float32),
        pltpu.VMEM((MSUP, H), jnp.float32),
        pltpu.VMEM_SHARED((NPAD, H), jnp.float32),
        pltpu.SemaphoreType.DMA,
        pltpu.SemaphoreType.DMA,
    ],
)
def _sc_scatter_add(m_hbm, dst3_hbm, zero_hbm, agg_hbm, di, bufa, bufb, acc, msa, msb):
    c = lax.axis_index("c")
    s = lax.axis_index("s")
    wid = c * NS + s
    base = wid * EPW
    r0 = s * RPT

    pltpu.sync_copy(zero_hbm.at[pl.ds(r0, RPT)], acc.at[pl.ds(r0, RPT)])
    pltpu.sync_copy(dst3_hbm.at[wid], di)
    plsc.subcore_barrier()

    def issue_load(t, buf, sem):
        pltpu.async_copy(m_hbm.at[pl.ds(base + t * MSUP, MSUP)], buf, sem)

    def wait_load(buf, sem):
        pltpu.make_async_copy(m_hbm.at[pl.ds(0, MSUP)], buf, sem).wait()

    def phase(t, buf, sem, obuf, osem):
        @pl.when(t + 1 < NMSUP)
        def _():
            issue_load(t + 1, obuf, osem)

        wait_load(buf, sem)
        for k in range(MSUB):
            pltpu.sync_copy(buf.at[pl.ds(k * CH, CH)],
                            acc.at[di.at[t * MSUB + k]], add=True)

    issue_load(0, bufa, msa)

    def body(o, carry):
        phase(2 * o, bufa, msa, bufb, msb)
        phase(2 * o + 1, bufb, msb, bufa, msa)
        return carry

    lax.fori_loop(0, NMSUP // 2, body, 0)
    phase(NMSUP - 1, bufa, msa, bufb, msb)

    plsc.subcore_barrier()
    pltpu.sync_copy(acc.at[pl.ds(r0, RPT)], agg_hbm.at[c].at[pl.ds(r0, RPT)])


BN = 2000
GN = N // BN
BE = 2000
GE = E // BE


def _emb_body(x_ref, emb_ref, h_ref):
    xb = x_ref[0, 0, :]
    onehot = (xb[:, None] == lax.broadcasted_iota(jnp.int32, (BN, NVOC), 1))
    h_ref[...] = jnp.dot(onehot.astype(jnp.float32), emb_ref[...],
                         preferred_element_type=jnp.float32)


_emb_call = pl.pallas_call(
    _emb_body,
    grid=(GN,),
    in_specs=[
        pl.BlockSpec((1, 1, BN), lambda i: (i, 0, 0)),
        pl.BlockSpec((NVOC, H), lambda i: (0, 0)),
    ],
    out_specs=pl.BlockSpec((BN, H), lambda i: (i, 0)),
    out_shape=jax.ShapeDtypeStruct((N, H), jnp.float32),
)


def _edge_body(zd_ref, zs_ref, ea_ref, wf_ref, bf_ref, ws_ref, bs_ref, m_ref):
    zd = zd_ref[...]
    zs = zs_ref[...]
    ea = ea_ref[...]
    f = (jnp.dot(zd, wf_ref[0:H, :], preferred_element_type=jnp.float32)
         + jnp.dot(zs, wf_ref[H:2 * H, :], preferred_element_type=jnp.float32)
         + jnp.dot(ea, wf_ref[2 * H:, :], preferred_element_type=jnp.float32)
         + bf_ref[...])
    s = (jnp.dot(zd, ws_ref[0:H, :], preferred_element_type=jnp.float32)
         + jnp.dot(zs, ws_ref[H:2 * H, :], preferred_element_type=jnp.float32)
         + jnp.dot(ea, ws_ref[2 * H:, :], preferred_element_type=jnp.float32)
         + bs_ref[...])
    sig = 1.0 / (1.0 + jnp.exp(-f))
    sp = jnp.maximum(s, 0.0) + jnp.log(1.0 + jnp.exp(-jnp.abs(s)))
    m_ref[...] = sig * sp


_edge_call = pl.pallas_call(
    _edge_body,
    grid=(GE,),
    in_specs=[
        pl.BlockSpec((BE, H), lambda i: (i, 0)),
        pl.BlockSpec((BE, H), lambda i: (i, 0)),
        pl.BlockSpec((BE, R), lambda i: (i, 0)),
        pl.BlockSpec((2 * H + R, H), lambda i: (0, 0)),
        pl.BlockSpec((H,), lambda i: (0,)),
        pl.BlockSpec((2 * H + R, H), lambda i: (0, 0)),
        pl.BlockSpec((H,), lambda i: (0,)),
    ],
    out_specs=pl.BlockSpec((BE, H), lambda i: (i, 0)),
    out_shape=jax.ShapeDtypeStruct((E, H), jnp.float32),
)


def _combine_body(h_ref, a0_ref, a1_ref, o_ref):
    o_ref[...] = jnp.maximum(h_ref[...] + a0_ref[...] + a1_ref[...], 0.0)


_combine_call = pl.pallas_call(
    _combine_body,
    grid=(GN,),
    in_specs=[
        pl.BlockSpec((BN, H), lambda i: (i, 0)),
        pl.BlockSpec((BN, H), lambda i: (i, 0)),
        pl.BlockSpec((BN, H), lambda i: (i, 0)),
    ],
    out_specs=pl.BlockSpec((BN, H), lambda i: (i, 0)),
    out_shape=jax.ShapeDtypeStruct((N, H), jnp.float32),
)


def _pool_body(h_ref, a0_ref, a1_ref, b_ref, wl_ref, bl_ref, o_ref, sums, cnts):
    i = pl.program_id(0)

    @pl.when(i == 0)
    def _():
        sums[...] = jnp.zeros_like(sums)
        cnts[...] = jnp.zeros_like(cnts)

    h3 = jnp.maximum(h_ref[...] + a0_ref[...] + a1_ref[...], 0.0)
    bb = b_ref[0, 0, :]
    onehot = (bb[:, None] == lax.broadcasted_iota(jnp.int32, (BN, G), 1)).astype(jnp.float32)
    sums[...] += lax.dot_general(onehot, h3, (((0,), (0,)), ((), ())),
                                 preferred_element_type=jnp.float32)
    cnts[...] += jnp.broadcast_to(jnp.sum(onehot, axis=0)[:, None], (G, H))

    @pl.when(i == GN - 1)
    def _():
        pooled = sums[...] / jnp.maximum(cnts[...], 1.0)
        o_ref[...] = jnp.dot(pooled, wl_ref[...],
                             preferred_element_type=jnp.float32) + bl_ref[...]


_pool_call = pl.pallas_call(
    _pool_body,
    grid=(GN,),
    in_specs=[
        pl.BlockSpec((BN, H), lambda i: (i, 0)),
        pl.BlockSpec((BN, H), lambda i: (i, 0)),
        pl.BlockSpec((BN, H), lambda i: (i, 0)),
        pl.BlockSpec((1, 1, BN), lambda i: (i, 0, 0)),
        pl.BlockSpec((H, H), lambda i: (0, 0)),
        pl.BlockSpec((H,), lambda i: (0,)),
    ],
    out_specs=pl.BlockSpec((G, H), lambda i: (0, 0)),
    out_shape=jax.ShapeDtypeStruct((G, H), jnp.float32),
    scratch_shapes=[
        pltpu.VMEM((G, H), jnp.float32),
        pltpu.VMEM((G, H), jnp.float32),
    ],
)


def kernel(x, edge_index, edge_attr, batch, emb,
           Wf1, bf1, Ws1, bs1, Wf2, bf2, Ws2, bs2, Wf3, bf3, Ws3, bs3, Wl, bl):
    src = edge_index[0].astype(jnp.int32)
    dst = edge_index[1].astype(jnp.int32)
    dst3 = dst.reshape(NW, NCHUNK, CH)
    x3 = x.reshape(GN, 1, BN).astype(jnp.int32)
    b3 = batch.reshape(GN, 1, BN).astype(jnp.int32)
    zero = jnp.zeros((NPAD, H), jnp.float32)

    h = _emb_call(x3, emb)
    layers = [(Wf1, bf1, Ws1, bs1), (Wf2, bf2, Ws2, bs2), (Wf3, bf3, Ws3, bs3)]
    agg = None
    for li, (Wf, bf, Ws, bs) in enumerate(layers):
        if li > 0:
            h = _combine_call(h, agg[0, :N], agg[1, :N])
        z = _sc_gather(h, src, dst)
        m = _edge_call(z[1], z[0], edge_attr, Wf, bf, Ws, bs)
        agg = _sc_scatter_add(m, dst3, zero)

    return _pool_call(h, agg[0, :N], agg[1, :N], b3, Wl, bl)
